# baseline (speedup 1.0000x reference)
import functools

import jax
import jax.numpy as jnp
import numpy as np
from jax import lax
from jax.experimental import pallas as pl
from jax.experimental.pallas import tpu as pltpu
from jax.experimental.pallas import tpu_sc as plsc

HUBERT_HOP = 320
HUBERT_SR = 16000
MEL_HOP = 512
MEL_SR = 44100
RATIO = (MEL_HOP / MEL_SR) / (HUBERT_HOP / HUBERT_SR)

N_FRAMES_STATIC = 28000
T_UNITS = 16384
C_DIM = 256

G = 16
NGROUPS = N_FRAMES_STATIC // G
NUM_WORKERS = 32
STEPS = -(-NGROUPS // NUM_WORKERS)
MAGIC = np.float32(8388608.0)
RATIO_F32 = np.float32(RATIO)


def _sc_body(units_hbm, dn_hbm, out_hbm, dn_v, idx_v, rows_v, tile_v, sem):
    core = lax.axis_index("c")
    subcore = lax.axis_index("s")
    wid = subcore * 2 + core

    pltpu.sync_copy(dn_hbm, dn_v)
    dn_f = dn_v[...].astype(jnp.float32)
    lanes = lax.broadcasted_iota(jnp.int32, (16,), 0)
    lanes_f = lanes.astype(jnp.float32)

    def step(k, _):
        g = k * NUM_WORKERS + wid

        @pl.when(g < NGROUPS)
        def _():
            t0 = (g * G).astype(jnp.float32)
            x = (t0 + lanes_f + dn_f) * RATIO_F32
            r = (x + MAGIC) - MAGIC
            idx = jnp.minimum(r.astype(jnp.int32), T_UNITS - 1)
            idx_v[...] = idx
            pltpu.async_copy(units_hbm.at[idx_v], rows_v, sem).wait()
            for c in range(C_DIM):
                col = plsc.load_gather(
                    rows_v, [lanes, jnp.full((16,), c, jnp.int32)]
                )
                tile_v[c] = col
            pltpu.sync_copy(tile_v, out_hbm.at[:, pl.ds(g * G, G)])

        return None

    lax.fori_loop(0, STEPS, step, None)


def kernel(units, n_frames):
    units2d = units.reshape(T_UNITS, C_DIM)
    dn = jnp.asarray(n_frames, jnp.int32) - N_FRAMES_STATIC
    dn_vec = jnp.broadcast_to(dn, (16,)).astype(jnp.int32)

    mesh = plsc.VectorSubcoreMesh(
        core_axis_name="c", subcore_axis_name="s", num_cores=2, num_subcores=16
    )
    sc_call = functools.partial(
        pl.kernel,
        out_type=jax.ShapeDtypeStruct((C_DIM, N_FRAMES_STATIC), jnp.float32),
        mesh=mesh,
        scratch_types=[
            pltpu.VMEM((16,), jnp.int32),
            pltpu.VMEM((16,), jnp.int32),
            pltpu.VMEM((G, C_DIM), jnp.float32),
            pltpu.VMEM((C_DIM, G), jnp.float32),
            pltpu.SemaphoreType.DMA,
        ],
        compiler_params=pltpu.CompilerParams(
            use_tc_tiling_on_sc=False, needs_layout_passes=False
        ),
    )(_sc_body)
    out2d = sc_call(units2d, dn_vec)
    return out2d.reshape(1, C_DIM, N_FRAMES_STATIC)
